# Initial kernel scaffold; baseline (speedup 1.0000x reference)
import jax
import jax.numpy as jnp
from jax.experimental import pallas as pl


def kernel(h, e, pos_enc, params, edge_index, batch):
    raise NotImplementedError("write your pallas kernel here")


# speedup vs baseline: 16.2554x; 16.2554x over previous
import functools
import numpy as np
import jax
import jax.numpy as jnp
from jax import lax
from jax.experimental import pallas as pl
from jax.experimental.pallas import tpu as pltpu

_HID = 96
_HEADS = 8
_DK = 12
_NG = 256
_OUT = 37
_EBLK = 2000
_NBLK = 1000
_ISQ = float(1.0 / np.sqrt(_DK))

_MH = np.kron(np.eye(_HEADS), np.ones((_DK, 1))).astype(np.float32)
_MT = np.kron(np.eye(_HEADS), np.ones((1, _DK))).astype(np.float32)
_P16 = np.concatenate([np.eye(_HEADS), np.zeros((_HEADS, 8))], 1).astype(np.float32)
_MT16 = np.concatenate([_MT, np.zeros((8, _HID))], 0).astype(np.float32)
_O96 = (np.ones((8, _HID)) / 8.0).astype(np.float32)


def _fullspec(shape):
    nd = len(shape)
    return pl.BlockSpec(shape, lambda *_: (0,) * nd)


def _dot(a, b):
    return jnp.dot(a, b, preferred_element_type=jnp.float32)


def _ln(x, g, b):
    m = jnp.mean(x, axis=-1, keepdims=True)
    xc = x - m
    v = jnp.mean(xc * xc, axis=-1, keepdims=True)
    return xc * lax.rsqrt(v + 1e-5) * g + b


def _embed_h_body(x_ref, w_ref, b_ref, pe_ref, o_ref):
    o_ref[...] = _dot(x_ref[...], w_ref[...]) + b_ref[...] + pe_ref[...]


def _embed_e_body(x_ref, w_ref, b_ref, o_ref):
    o_ref[...] = _dot(x_ref[...], w_ref[...]) + b_ref[...]


def _qkv_body(x_ref, wq, bq, wk, bk, wv, bv, q_o, k_o, v_o):
    x = x_ref[...]
    q_o[...] = _dot(x, wq[...]) + bq[...]
    k_o[...] = _dot(x, wk[...]) + bk[...]
    v_o[...] = _dot(x, wv[...]) + bv[...]


def _edge_body(ee_ref, ke_ref, qe_ref, ve_ref,
               wep, bep, woe, boe, w1, b1, w2, b2,
               g1, gb1, g2, gb2, mh, mt, p16,
               ee_o, wv_o, w_o):
    ee = ee_ref[...]
    ep = _dot(ee, wep[...]) + bep[...]
    score = ke_ref[...] * qe_ref[...] * ep * _ISQ
    s = _dot(score, mh[...])
    w = jnp.exp(jnp.clip(s, -5.0, 5.0))
    wv_o[...] = _dot(w, mt[...]) * ve_ref[...]
    w_o[...] = _dot(w, p16[...])
    oe = _dot(score, woe[...]) + boe[...]
    e1 = _ln(ee + oe, g1[...], gb1[...])
    f = jnp.maximum(_dot(e1, w1[...]) + b1[...], 0.0)
    f = _dot(f, w2[...]) + b2[...]
    ee_o[...] = _ln(e1 + f, g2[...], gb2[...])


def _node_body(hh_ref, nump_ref, denp_ref,
               mt16, wo, bo, w1, b1, w2, b2,
               g1, gb1, g2, gb2, out_ref):
    num = jnp.sum(nump_ref[...], axis=0)
    den16 = jnp.sum(denp_ref[...], axis=0)
    denf = _dot(den16, mt16[...])
    h_attn = num / (denf + 1e-6)
    hh = hh_ref[...]
    o = _dot(h_attn, wo[...]) + bo[...]
    h1 = _ln(hh + o, g1[...], gb1[...])
    f = jnp.maximum(_dot(h1, w1[...]) + b1[...], 0.0)
    f = _dot(f, w2[...]) + b2[...]
    out_ref[...] = _ln(h1 + f, g2[...], gb2[...])


def _pool_body(batch_ref, hh_ref, sums_ref, cnts_ref):
    i = pl.program_id(0)

    @pl.when(i == 0)
    def _():
        sums_ref[...] = jnp.zeros_like(sums_ref)
        cnts_ref[...] = jnp.zeros_like(cnts_ref)

    b = batch_ref[0, 0, :]
    oh = (b[:, None] == lax.broadcasted_iota(jnp.int32, (_NBLK, _NG), 1))
    oh = oh.astype(jnp.float32)
    sums_ref[...] += lax.dot_general(oh, hh_ref[...], (((0,), (0,)), ((), ())),
                                     preferred_element_type=jnp.float32)
    cnts_ref[...] += lax.dot_general(oh, jnp.ones((_NBLK, 8), jnp.float32),
                                     (((0,), (0,)), ((), ())),
                                     preferred_element_type=jnp.float32)


def _mlp_body(sums_ref, cnts_ref, o96, w1, b1, w2, b2, w3, b3, y_ref):
    cnt96 = _dot(cnts_ref[...], o96[...])
    hg = sums_ref[...] / jnp.maximum(cnt96, 1.0)
    y = jnp.maximum(_dot(hg, w1[...]) + b1[...], 0.0)
    y = jnp.maximum(_dot(y, w2[...]) + b2[...], 0.0)
    y_ref[...] = _dot(y, w3[...]) + b3[...]


def _row1(b):
    return b.reshape(1, -1)


def _embed_h(h8, w8, b, pos_enc, n):
    return pl.pallas_call(
        _embed_h_body,
        grid=(n // _NBLK,),
        in_specs=[pl.BlockSpec((_NBLK, 8), lambda i: (i, 0)),
                  _fullspec(w8.shape), _fullspec(b.shape),
                  pl.BlockSpec((_NBLK, _HID), lambda i: (i, 0))],
        out_specs=pl.BlockSpec((_NBLK, _HID), lambda i: (i, 0)),
        out_shape=jax.ShapeDtypeStruct((n, _HID), jnp.float32),
    )(h8, w8, b, pos_enc)


def _embed_e(e8, w8, b, m):
    return pl.pallas_call(
        _embed_e_body,
        grid=(m // _EBLK,),
        in_specs=[pl.BlockSpec((_EBLK, 8), lambda i: (i, 0)),
                  _fullspec(w8.shape), _fullspec(b.shape)],
        out_specs=pl.BlockSpec((_EBLK, _HID), lambda i: (i, 0)),
        out_shape=jax.ShapeDtypeStruct((m, _HID), jnp.float32),
    )(e8, w8, b)


def _qkv(hh, lp, n):
    args = [hh, lp['Q'][0], _row1(lp['Q'][1]), lp['K'][0], _row1(lp['K'][1]),
            lp['V'][0], _row1(lp['V'][1])]
    blk = pl.BlockSpec((_NBLK, _HID), lambda i: (i, 0))
    return pl.pallas_call(
        _qkv_body,
        grid=(n // _NBLK,),
        in_specs=[blk] + [_fullspec(a.shape) for a in args[1:]],
        out_specs=(blk, blk, blk),
        out_shape=tuple(jax.ShapeDtypeStruct((n, _HID), jnp.float32)
                        for _ in range(3)),
    )(*args)


def _edge_stage(ee, ke, qe, ve, lp, consts, m):
    mh, mt, p16 = consts
    args = [ee, ke, qe, ve,
            lp['Ep'][0], _row1(lp['Ep'][1]), lp['Oe'][0], _row1(lp['Oe'][1]),
            lp['ffn_e1'][0], _row1(lp['ffn_e1'][1]),
            lp['ffn_e2'][0], _row1(lp['ffn_e2'][1]),
            _row1(lp['ln1_e'][0]), _row1(lp['ln1_e'][1]),
            _row1(lp['ln2_e'][0]), _row1(lp['ln2_e'][1]),
            mh, mt, p16]
    blk = pl.BlockSpec((_EBLK, _HID), lambda i: (i, 0))
    blk16 = pl.BlockSpec((_EBLK, 16), lambda i: (i, 0))
    return pl.pallas_call(
        _edge_body,
        grid=(m // _EBLK,),
        in_specs=[blk, blk, blk, blk] + [_fullspec(a.shape) for a in args[4:]],
        out_specs=(blk, blk, blk16),
        out_shape=(jax.ShapeDtypeStruct((m, _HID), jnp.float32),
                   jax.ShapeDtypeStruct((m, _HID), jnp.float32),
                   jax.ShapeDtypeStruct((m, 16), jnp.float32)),
    )(*args)


def _node_stage(hh, nump, denp, lp, mt16, n):
    s = nump.shape[0]
    args = [hh, nump, denp, mt16,
            lp['O'][0], _row1(lp['O'][1]),
            lp['ffn_h1'][0], _row1(lp['ffn_h1'][1]),
            lp['ffn_h2'][0], _row1(lp['ffn_h2'][1]),
            _row1(lp['ln1_h'][0]), _row1(lp['ln1_h'][1]),
            _row1(lp['ln2_h'][0]), _row1(lp['ln2_h'][1])]
    blk = pl.BlockSpec((_NBLK, _HID), lambda i: (i, 0))
    return pl.pallas_call(
        _node_body,
        grid=(n // _NBLK,),
        in_specs=[blk,
                  pl.BlockSpec((s, _NBLK, _HID), lambda i: (0, i, 0)),
                  pl.BlockSpec((s, _NBLK, 16), lambda i: (0, i, 0))]
                 + [_fullspec(a.shape) for a in args[3:]],
        out_specs=blk,
        out_shape=jax.ShapeDtypeStruct((n, _HID), jnp.float32),
    )(*args)


def _readout(hh, batch, params, o96, n):
    batch3 = batch.reshape(n // _NBLK, 1, _NBLK)
    sums, cnts = pl.pallas_call(
        _pool_body,
        grid=(n // _NBLK,),
        in_specs=[pl.BlockSpec((1, 1, _NBLK), lambda i: (i, 0, 0)),
                  pl.BlockSpec((_NBLK, _HID), lambda i: (i, 0))],
        out_specs=(pl.BlockSpec((_NG, _HID), lambda i: (0, 0)),
                   pl.BlockSpec((_NG, 8), lambda i: (0, 0))),
        out_shape=(jax.ShapeDtypeStruct((_NG, _HID), jnp.float32),
                   jax.ShapeDtypeStruct((_NG, 8), jnp.float32)),
    )(batch3, hh)
    args = [sums, cnts, o96,
            params['mlp1'][0], _row1(params['mlp1'][1]),
            params['mlp2'][0], _row1(params['mlp2'][1]),
            params['mlp3'][0], _row1(params['mlp3'][1])]
    return pl.pallas_call(
        _mlp_body,
        in_specs=[_fullspec(a.shape) for a in args],
        out_specs=_fullspec((_NG, _OUT)),
        out_shape=jax.ShapeDtypeStruct((_NG, _OUT), jnp.float32),
    )(*args)


def kernel(h, e, pos_enc, params, edge_index, batch):
    n = h.shape[0]
    m = e.shape[0]
    src = edge_index[0]
    dst = edge_index[1]

    mh = jnp.asarray(_MH)
    mt = jnp.asarray(_MT)
    p16 = jnp.asarray(_P16)
    mt16 = jnp.asarray(_MT16)
    o96 = jnp.asarray(_O96)

    h8 = jnp.pad(h, ((0, 0), (0, 1)))
    e8 = jnp.pad(e, ((0, 0), (0, 1)))
    wh8 = jnp.pad(params['emb_h'][0], ((0, 1), (0, 0)))
    we8 = jnp.pad(params['emb_e'][0], ((0, 1), (0, 0)))

    hh = _embed_h(h8, wh8, _row1(params['emb_h'][1]), pos_enc, n)
    ee = _embed_e(e8, we8, _row1(params['emb_e'][1]), m)

    for lp in params['layers']:
        q, k, v = _qkv(hh, lp, n)
        ke = jnp.take(k, src, axis=0)
        qe = jnp.take(q, dst, axis=0)
        ve = jnp.take(v, src, axis=0)
        ee, wv, w16 = _edge_stage(ee, ke, qe, ve, lp, (mh, mt, p16), m)
        num = jax.ops.segment_sum(wv, dst, num_segments=n)
        den16 = jax.ops.segment_sum(w16, dst, num_segments=n)
        hh = _node_stage(hh, num[None], den16[None], lp, mt16, n)

    return _readout(hh, batch, params, o96, n)


# speedup vs baseline: 33.7132x; 2.0740x over previous
import functools
import numpy as np
import jax
import jax.numpy as jnp
from jax import lax
from jax.experimental import pallas as pl
from jax.experimental.pallas import tpu as pltpu
from jax.experimental.pallas import tpu_sc as plsc

_HID = 96
_HEADS = 8
_DK = 12
_NG = 256
_OUT = 37
_EBLK = 2000
_NBLK = 1000
_ISQ = float(1.0 / np.sqrt(_DK))

_WID = 128
_CH = 400
_NW = 32
_RNG = 9728

_MH = np.kron(np.eye(_HEADS), np.ones((_DK, 1))).astype(np.float32)
_MT = np.kron(np.eye(_HEADS), np.ones((1, _DK))).astype(np.float32)
_MH128 = np.pad(_MH, ((0, _WID - _HID), (0, 0)))
_MT128 = np.pad(_MT, ((0, 0), (0, _WID - _HID)))
_PW = np.zeros((_HEADS, _WID), np.float32)
for _h in range(_HEADS):
    _PW[_h, _HID + _h] = 1.0
_SEL96 = np.pad(np.eye(_HID, dtype=np.float32), ((0, _WID - _HID), (0, 0)))
_MT8 = np.zeros((_WID, _HID), np.float32)
for _h in range(_HEADS):
    _MT8[_HID + _h, _h * _DK:(_h + 1) * _DK] = 1.0
_O96 = (np.ones((8, _HID)) / 8.0).astype(np.float32)


def _fullspec(shape):
    nd = len(shape)
    return pl.BlockSpec(shape, lambda *_: (0,) * nd)


def _dot(a, b):
    return jnp.dot(a, b, preferred_element_type=jnp.float32)


def _ln(x, g, b):
    m = jnp.mean(x, axis=-1, keepdims=True)
    xc = x - m
    v = jnp.mean(xc * xc, axis=-1, keepdims=True)
    return xc * lax.rsqrt(v + 1e-5) * g + b


def _row1(b):
    return b.reshape(1, -1)


def _padc(w):
    return jnp.pad(w, ((0, 0), (0, _WID - w.shape[1])))


def _padr(w):
    return jnp.pad(w, ((0, _WID - w.shape[0]), (0, 0)))


def _embed_h_body(x_ref, w_ref, b_ref, pe_ref, o_ref):
    o_ref[...] = _dot(x_ref[...], w_ref[...]) + b_ref[...] + pe_ref[...]


def _embed_e_body(x_ref, w_ref, b_ref, o_ref):
    o_ref[...] = _dot(x_ref[...], w_ref[...]) + b_ref[...]


def _qkv_body(x_ref, wq, bq, wk, bk, wv, bv, q_o, k_o, v_o):
    x = x_ref[...]
    q_o[...] = _dot(x, wq[...]) + bq[...]
    k_o[...] = _dot(x, wk[...]) + bk[...]
    v_o[...] = _dot(x, wv[...]) + bv[...]


def _edge_body(ee_ref, ke_ref, qe_ref, ve_ref,
               wep, bep, woe, boe, w1, b1, w2, b2,
               g1, gb1, g2, gb2, mh, mt, pw,
               ee_o, wvw_o):
    ee = ee_ref[...]
    ep = _dot(ee, wep[...]) + bep[...]
    score = ke_ref[...] * qe_ref[...] * ep * _ISQ
    s = _dot(score, mh[...])
    w = jnp.exp(jnp.clip(s, -5.0, 5.0))
    wvw_o[...] = _dot(w, mt[...]) * ve_ref[...] + _dot(w, pw[...])
    oe = _dot(score, woe[...]) + boe[...]
    e1 = _ln(ee + oe, g1[...], gb1[...])
    f = jnp.maximum(_dot(e1, w1[...]) + b1[...], 0.0)
    f = _dot(f, w2[...]) + b2[...]
    ee_o[...] = _ln(e1 + f, g2[...], gb2[...])


def _node_body(hh_ref, nump_ref, sel96, mt8, wo, bo, w1, b1, w2, b2,
               g1, gb1, g2, gb2, out_ref):
    nd = jnp.sum(nump_ref[...], axis=0)
    num = _dot(nd, sel96[...])
    denf = _dot(nd, mt8[...])
    h_attn = num / (denf + 1e-6)
    hh = hh_ref[...]
    o = _dot(h_attn, wo[...]) + bo[...]
    h1 = _ln(hh + o, g1[...], gb1[...])
    f = jnp.maximum(_dot(h1, w1[...]) + b1[...], 0.0)
    f = _dot(f, w2[...]) + b2[...]
    out_ref[...] = _ln(h1 + f, g2[...], gb2[...])


def _pool_body(batch_ref, hh_ref, sums_ref, cnts_ref):
    i = pl.program_id(0)

    @pl.when(i == 0)
    def _():
        sums_ref[...] = jnp.zeros_like(sums_ref)
        cnts_ref[...] = jnp.zeros_like(cnts_ref)

    b = batch_ref[0, 0, :]
    oh = (b[:, None] == lax.broadcasted_iota(jnp.int32, (_NBLK, _NG), 1))
    oh = oh.astype(jnp.float32)
    sums_ref[...] += lax.dot_general(oh, hh_ref[...], (((0,), (0,)), ((), ())),
                                     preferred_element_type=jnp.float32)
    cnts_ref[...] += lax.dot_general(oh, jnp.ones((_NBLK, 8), jnp.float32),
                                     (((0,), (0,)), ((), ())),
                                     preferred_element_type=jnp.float32)


def _mlp_body(sums_ref, cnts_ref, o96, w1, b1, w2, b2, w3, b3, y_ref):
    cnt96 = _dot(cnts_ref[...], o96[...])
    hg = sums_ref[...] / jnp.maximum(cnt96, 1.0)
    y = jnp.maximum(_dot(hg, w1[...]) + b1[...], 0.0)
    y = jnp.maximum(_dot(y, w2[...]) + b2[...], 0.0)
    y_ref[...] = _dot(y, w3[...]) + b3[...]


def _sc_gather(k, q, v, src, dst, n, m):
    mesh = plsc.VectorSubcoreMesh(core_axis_name="c", subcore_axis_name="s")
    chunks = m // _CH
    witers = (chunks + _NW - 1) // _NW

    @functools.partial(
        pl.kernel, mesh=mesh,
        out_type=(jax.ShapeDtypeStruct((m, _WID), jnp.float32),
                  jax.ShapeDtypeStruct((m, _WID), jnp.float32),
                  jax.ShapeDtypeStruct((m, _WID), jnp.float32)),
        scratch_types=[pltpu.VMEM((_CH,), jnp.int32),
                       pltpu.VMEM((_CH, _WID), jnp.float32),
                       pltpu.SemaphoreType.DMA],
    )
    def gk(k_hbm, q_hbm, v_hbm, src_hbm, dst_hbm, ke_o, qe_o, ve_o,
           idx_v, rows_v, sem):
        c = lax.axis_index("c")
        s = lax.axis_index("s")
        w = s * 2 + c

        def body(t, carry):
            chunk = t * _NW + w

            @pl.when(chunk < chunks)
            def _():
                base = pl.multiple_of(chunk * _CH, 8)
                pltpu.sync_copy(src_hbm.at[pl.ds(base, _CH)], idx_v)
                pltpu.async_copy(k_hbm.at[idx_v], rows_v, sem).wait()
                pltpu.sync_copy(rows_v, ke_o.at[pl.ds(base, _CH)])
                pltpu.async_copy(v_hbm.at[idx_v], rows_v, sem).wait()
                pltpu.sync_copy(rows_v, ve_o.at[pl.ds(base, _CH)])
                pltpu.sync_copy(dst_hbm.at[pl.ds(base, _CH)], idx_v)
                pltpu.async_copy(q_hbm.at[idx_v], rows_v, sem).wait()
                pltpu.sync_copy(rows_v, qe_o.at[pl.ds(base, _CH)])

            return carry

        lax.fori_loop(0, witers, body, 0)

    return gk(k, q, v, src, dst)


def _sc_scatter(wvw, dst, zeros, n, m):
    mesh = plsc.VectorSubcoreMesh(core_axis_name="c", subcore_axis_name="s")
    chunks = m // _CH
    witers = (chunks + _NW - 1) // _NW
    npass = (n + _RNG - 1) // _RNG
    n_out = npass * _RNG
    rows_t = _RNG // 16

    @functools.partial(
        pl.kernel, mesh=mesh,
        out_type=jax.ShapeDtypeStruct((2, n_out, _WID), jnp.float32),
        scratch_types=[pltpu.VMEM((_CH,), jnp.int32),
                       pltpu.VMEM((_CH,), jnp.int32),
                       pltpu.VMEM((_CH, _WID), jnp.float32),
                       pltpu.VMEM_SHARED((_RNG + 8, _WID), jnp.float32)],
    )
    def sk(wvw_hbm, dst_hbm, z_hbm, nump_o, idx_v, idx2_v, buf_v, acc):
        c = lax.axis_index("c")
        s = lax.axis_index("s")
        w = s * 2 + c
        r0 = pl.multiple_of(s * rows_t, 8)
        for p in range(npass):
            lo = p * _RNG
            pltpu.sync_copy(z_hbm.at[pl.ds(r0, rows_t)],
                            acc.at[pl.ds(r0, rows_t)])

            @pl.when(s == 0)
            def _():
                pltpu.sync_copy(z_hbm.at[pl.ds(_RNG, 8)],
                                acc.at[pl.ds(_RNG, 8)])

            plsc.subcore_barrier()

            def body(t, carry, lo=lo):
                chunk = t * _NW + w

                @pl.when(chunk < chunks)
                def _():
                    base = pl.multiple_of(chunk * _CH, 8)
                    pltpu.sync_copy(dst_hbm.at[pl.ds(base, _CH)], idx_v)
                    pltpu.sync_copy(wvw_hbm.at[pl.ds(base, _CH)], buf_v)

                    def vbody(j, cc):
                        o = pl.multiple_of(j * 16, 8)
                        rel = idx_v[pl.ds(o, 16)] - lo
                        ok = (rel >= 0) & (rel < _RNG)
                        idx2_v[pl.ds(o, 16)] = jnp.where(ok, rel, _RNG)
                        return cc

                    lax.fori_loop(0, _CH // 16, vbody, 0)
                    pltpu.sync_copy(buf_v, acc.at[idx2_v], add=True)

                return carry

            lax.fori_loop(0, witers, body, 0)
            plsc.subcore_barrier()
            pltpu.sync_copy(acc.at[pl.ds(r0, rows_t)],
                            nump_o.at[c, pl.ds(pl.multiple_of(lo + r0, 8),
                                               rows_t)])

    return sk(wvw, dst, zeros)


def _embed_h(h8, w8, b, pos_enc, n):
    return pl.pallas_call(
        _embed_h_body,
        grid=(n // _NBLK,),
        in_specs=[pl.BlockSpec((_NBLK, 8), lambda i: (i, 0)),
                  _fullspec(w8.shape), _fullspec(b.shape),
                  pl.BlockSpec((_NBLK, _HID), lambda i: (i, 0))],
        out_specs=pl.BlockSpec((_NBLK, _HID), lambda i: (i, 0)),
        out_shape=jax.ShapeDtypeStruct((n, _HID), jnp.float32),
    )(h8, w8, b, pos_enc)


def _embed_e(e8, w8, b, m):
    return pl.pallas_call(
        _embed_e_body,
        grid=(m // _EBLK,),
        in_specs=[pl.BlockSpec((_EBLK, 8), lambda i: (i, 0)),
                  _fullspec(w8.shape), _fullspec(b.shape)],
        out_specs=pl.BlockSpec((_EBLK, _HID), lambda i: (i, 0)),
        out_shape=jax.ShapeDtypeStruct((m, _HID), jnp.float32),
    )(e8, w8, b)


def _qkv(hh, lp, n):
    args = [hh,
            _padc(lp['Q'][0]), _padc(_row1(lp['Q'][1])),
            _padc(lp['K'][0]), _padc(_row1(lp['K'][1])),
            _padc(lp['V'][0]), _padc(_row1(lp['V'][1]))]
    blk = pl.BlockSpec((_NBLK, _HID), lambda i: (i, 0))
    blkw = pl.BlockSpec((_NBLK, _WID), lambda i: (i, 0))
    return pl.pallas_call(
        _qkv_body,
        grid=(n // _NBLK,),
        in_specs=[blk] + [_fullspec(a.shape) for a in args[1:]],
        out_specs=(blkw, blkw, blkw),
        out_shape=tuple(jax.ShapeDtypeStruct((n, _WID), jnp.float32)
                        for _ in range(3)),
    )(*args)


def _edge_stage(ee, ke, qe, ve, lp, consts, m):
    mh, mt, pw = consts
    args = [ee, ke, qe, ve,
            _padc(lp['Ep'][0]), _padc(_row1(lp['Ep'][1])),
            _padr(lp['Oe'][0]), _row1(lp['Oe'][1]),
            lp['ffn_e1'][0], _row1(lp['ffn_e1'][1]),
            lp['ffn_e2'][0], _row1(lp['ffn_e2'][1]),
            _row1(lp['ln1_e'][0]), _row1(lp['ln1_e'][1]),
            _row1(lp['ln2_e'][0]), _row1(lp['ln2_e'][1]),
            mh, mt, pw]
    blk = pl.BlockSpec((_EBLK, _HID), lambda i: (i, 0))
    blkw = pl.BlockSpec((_EBLK, _WID), lambda i: (i, 0))
    return pl.pallas_call(
        _edge_body,
        grid=(m // _EBLK,),
        in_specs=[blk, blkw, blkw, blkw] + [_fullspec(a.shape) for a in args[4:]],
        out_specs=(blk, blkw),
        out_shape=(jax.ShapeDtypeStruct((m, _HID), jnp.float32),
                   jax.ShapeDtypeStruct((m, _WID), jnp.float32)),
    )(*args)


def _node_stage(hh, nump, lp, sel96, mt8, n):
    s = nump.shape[0]
    args = [hh, nump, sel96, mt8,
            lp['O'][0], _row1(lp['O'][1]),
            lp['ffn_h1'][0], _row1(lp['ffn_h1'][1]),
            lp['ffn_h2'][0], _row1(lp['ffn_h2'][1]),
            _row1(lp['ln1_h'][0]), _row1(lp['ln1_h'][1]),
            _row1(lp['ln2_h'][0]), _row1(lp['ln2_h'][1])]
    blk = pl.BlockSpec((_NBLK, _HID), lambda i: (i, 0))
    return pl.pallas_call(
        _node_body,
        grid=(n // _NBLK,),
        in_specs=[blk,
                  pl.BlockSpec((s, _NBLK, _WID), lambda i: (0, i, 0))]
                 + [_fullspec(a.shape) for a in args[2:]],
        out_specs=blk,
        out_shape=jax.ShapeDtypeStruct((n, _HID), jnp.float32),
    )(*args)


def _readout(hh, batch, params, o96, n):
    batch3 = batch.reshape(n // _NBLK, 1, _NBLK)
    sums, cnts = pl.pallas_call(
        _pool_body,
        grid=(n // _NBLK,),
        in_specs=[pl.BlockSpec((1, 1, _NBLK), lambda i: (i, 0, 0)),
                  pl.BlockSpec((_NBLK, _HID), lambda i: (i, 0))],
        out_specs=(pl.BlockSpec((_NG, _HID), lambda i: (0, 0)),
                   pl.BlockSpec((_NG, 8), lambda i: (0, 0))),
        out_shape=(jax.ShapeDtypeStruct((_NG, _HID), jnp.float32),
                   jax.ShapeDtypeStruct((_NG, 8), jnp.float32)),
    )(batch3, hh)
    args = [sums, cnts, o96,
            params['mlp1'][0], _row1(params['mlp1'][1]),
            params['mlp2'][0], _row1(params['mlp2'][1]),
            params['mlp3'][0], _row1(params['mlp3'][1])]
    return pl.pallas_call(
        _mlp_body,
        in_specs=[_fullspec(a.shape) for a in args],
        out_specs=_fullspec((_NG, _OUT)),
        out_shape=jax.ShapeDtypeStruct((_NG, _OUT), jnp.float32),
    )(*args)


def kernel(h, e, pos_enc, params, edge_index, batch):
    n = h.shape[0]
    m = e.shape[0]
    src = edge_index[0]
    dst = edge_index[1]

    mh = jnp.asarray(_MH128)
    mt = jnp.asarray(_MT128)
    pw = jnp.asarray(_PW)
    sel96 = jnp.asarray(_SEL96)
    mt8 = jnp.asarray(_MT8)
    o96 = jnp.asarray(_O96)

    h8 = jnp.pad(h, ((0, 0), (0, 1)))
    e8 = jnp.pad(e, ((0, 0), (0, 1)))
    wh8 = jnp.pad(params['emb_h'][0], ((0, 1), (0, 0)))
    we8 = jnp.pad(params['emb_e'][0], ((0, 1), (0, 0)))

    hh = _embed_h(h8, wh8, _row1(params['emb_h'][1]), pos_enc, n)
    ee = _embed_e(e8, we8, _row1(params['emb_e'][1]), m)

    zeros = jnp.zeros((_RNG + 8, _WID), jnp.float32)
    for lp in params['layers']:
        q, k, v = _qkv(hh, lp, n)
        ke, qe, ve = _sc_gather(k, q, v, src, dst, n, m)
        ee, wvw = _edge_stage(ee, ke, qe, ve, lp, (mh, mt, pw), m)
        nump = _sc_scatter(wvw, dst, zeros, n, m)
        hh = _node_stage(hh, nump, lp, sel96, mt8, n)

    return _readout(hh, batch, params, o96, n)


# speedup vs baseline: 38.8698x; 1.1530x over previous
import functools
import numpy as np
import jax
import jax.numpy as jnp
from jax import lax
from jax.experimental import pallas as pl
from jax.experimental.pallas import tpu as pltpu
from jax.experimental.pallas import tpu_sc as plsc

_HID = 96
_HEADS = 8
_DK = 12
_NG = 256
_OUT = 37
_EBLK = 2000
_NBLK = 1000
_ISQ = float(1.0 / np.sqrt(_DK))

_WID = 128
_CHG = 400
_CHS = 160
_NW = 32
_RNG = 11008

_MH = np.kron(np.eye(_HEADS), np.ones((_DK, 1))).astype(np.float32)
_MT = np.kron(np.eye(_HEADS), np.ones((1, _DK))).astype(np.float32)
_MH128 = np.pad(_MH, ((0, _WID - _HID), (0, 0)))
_MT128 = np.pad(_MT, ((0, 0), (0, _WID - _HID)))
_PW = np.zeros((_HEADS, _WID), np.float32)
for _h in range(_HEADS):
    _PW[_h, _HID + _h] = 1.0
_SEL96 = np.pad(np.eye(_HID, dtype=np.float32), ((0, _WID - _HID), (0, 0)))
_MT8 = np.zeros((_WID, _HID), np.float32)
for _h in range(_HEADS):
    _MT8[_HID + _h, _h * _DK:(_h + 1) * _DK] = 1.0
_O96 = (np.ones((8, _HID)) / 8.0).astype(np.float32)


def _fullspec(shape):
    nd = len(shape)
    return pl.BlockSpec(shape, lambda *_: (0,) * nd)


def _dot(a, b):
    return jnp.dot(a, b, preferred_element_type=jnp.float32)


def _ln(x, g, b):
    m = jnp.mean(x, axis=-1, keepdims=True)
    xc = x - m
    v = jnp.mean(xc * xc, axis=-1, keepdims=True)
    return xc * lax.rsqrt(v + 1e-5) * g + b


def _row1(b):
    return b.reshape(1, -1)


def _padc(w):
    return jnp.pad(w, ((0, 0), (0, _WID - w.shape[1])))


def _padr(w):
    return jnp.pad(w, ((0, _WID - w.shape[0]), (0, 0)))


def _embed_h_body(x_ref, w_ref, b_ref, pe_ref, o_ref):
    o_ref[...] = _dot(x_ref[...], w_ref[...]) + b_ref[...] + pe_ref[...]


def _embed_e_body(x_ref, w_ref, b_ref, o_ref):
    o_ref[...] = _dot(x_ref[...], w_ref[...]) + b_ref[...]


def _qkv_body(x_ref, wq, bq, wk, bk, wv, bv, q_o, k_o, v_o):
    x = x_ref[...]
    q_o[...] = _dot(x, wq[...]) + bq[...]
    k_o[...] = _dot(x, wk[...]) + bk[...]
    v_o[...] = _dot(x, wv[...]) + bv[...]


def _edge_body(ee_ref, ke_ref, qe_ref, ve_ref,
               wep, bep, woe, boe, w1, b1, w2, b2,
               g1, gb1, g2, gb2, mh, mt, pw,
               ee_o, wvw_o):
    ee = ee_ref[...]
    ep = _dot(ee, wep[...]) + bep[...]
    score = ke_ref[...] * qe_ref[...] * ep * _ISQ
    s = _dot(score, mh[...])
    w = jnp.exp(jnp.clip(s, -5.0, 5.0))
    wvw_o[...] = _dot(w, mt[...]) * ve_ref[...] + _dot(w, pw[...])
    oe = _dot(score, woe[...]) + boe[...]
    e1 = _ln(ee + oe, g1[...], gb1[...])
    f = jnp.maximum(_dot(e1, w1[...]) + b1[...], 0.0)
    f = _dot(f, w2[...]) + b2[...]
    ee_o[...] = _ln(e1 + f, g2[...], gb2[...])


def _node_body(hh_ref, nump_ref, sel96, mt8, wo, bo, w1, b1, w2, b2,
               g1, gb1, g2, gb2, out_ref):
    nd = jnp.sum(nump_ref[...], axis=0)
    num = _dot(nd, sel96[...])
    denf = _dot(nd, mt8[...])
    h_attn = num / (denf + 1e-6)
    hh = hh_ref[...]
    o = _dot(h_attn, wo[...]) + bo[...]
    h1 = _ln(hh + o, g1[...], gb1[...])
    f = jnp.maximum(_dot(h1, w1[...]) + b1[...], 0.0)
    f = _dot(f, w2[...]) + b2[...]
    out_ref[...] = _ln(h1 + f, g2[...], gb2[...])


def _pool_body(batch_ref, hh_ref, sums_ref, cnts_ref):
    i = pl.program_id(0)

    @pl.when(i == 0)
    def _():
        sums_ref[...] = jnp.zeros_like(sums_ref)
        cnts_ref[...] = jnp.zeros_like(cnts_ref)

    b = batch_ref[0, 0, :]
    oh = (b[:, None] == lax.broadcasted_iota(jnp.int32, (_NBLK, _NG), 1))
    oh = oh.astype(jnp.float32)
    sums_ref[...] += lax.dot_general(oh, hh_ref[...], (((0,), (0,)), ((), ())),
                                     preferred_element_type=jnp.float32)
    cnts_ref[...] += lax.dot_general(oh, jnp.ones((_NBLK, 8), jnp.float32),
                                     (((0,), (0,)), ((), ())),
                                     preferred_element_type=jnp.float32)


def _mlp_body(sums_ref, cnts_ref, o96, w1, b1, w2, b2, w3, b3, y_ref):
    cnt96 = _dot(cnts_ref[...], o96[...])
    hg = sums_ref[...] / jnp.maximum(cnt96, 1.0)
    y = jnp.maximum(_dot(hg, w1[...]) + b1[...], 0.0)
    y = jnp.maximum(_dot(y, w2[...]) + b2[...], 0.0)
    y_ref[...] = _dot(y, w3[...]) + b3[...]


def _sc_gather(k, q, v, src, dst, n, m):
    mesh = plsc.VectorSubcoreMesh(core_axis_name="c", subcore_axis_name="s")
    chunks = m // _CHG
    witers = (chunks + _NW - 1) // _NW

    @functools.partial(
        pl.kernel, mesh=mesh,
        out_type=(jax.ShapeDtypeStruct((m, _WID), jnp.float32),
                  jax.ShapeDtypeStruct((m, _WID), jnp.float32),
                  jax.ShapeDtypeStruct((m, _WID), jnp.float32)),
        scratch_types=[pltpu.VMEM((_CHG,), jnp.int32),
                       pltpu.VMEM((_CHG, _WID), jnp.float32),
                       pltpu.SemaphoreType.DMA],
    )
    def gk(k_hbm, q_hbm, v_hbm, src_hbm, dst_hbm, ke_o, qe_o, ve_o,
           idx_v, rows_v, sem):
        c = lax.axis_index("c")
        s = lax.axis_index("s")
        w = s * 2 + c

        def body(t, carry):
            chunk = t * _NW + w

            @pl.when(chunk < chunks)
            def _():
                base = pl.multiple_of(chunk * _CHG, 8)
                pltpu.sync_copy(src_hbm.at[pl.ds(base, _CHG)], idx_v)
                pltpu.async_copy(k_hbm.at[idx_v], rows_v, sem).wait()
                pltpu.sync_copy(rows_v, ke_o.at[pl.ds(base, _CHG)])
                pltpu.async_copy(v_hbm.at[idx_v], rows_v, sem).wait()
                pltpu.sync_copy(rows_v, ve_o.at[pl.ds(base, _CHG)])
                pltpu.sync_copy(dst_hbm.at[pl.ds(base, _CHG)], idx_v)
                pltpu.async_copy(q_hbm.at[idx_v], rows_v, sem).wait()
                pltpu.sync_copy(rows_v, qe_o.at[pl.ds(base, _CHG)])

            return carry

        lax.fori_loop(0, witers, body, 0)

    return gk(k, q, v, src, dst)


def _sc_scatter(wvw, dst, zeros, n, m):
    mesh = plsc.VectorSubcoreMesh(core_axis_name="c", subcore_axis_name="s")
    chunks = m // _CHS
    witers = (chunks + _NW - 1) // _NW
    pairs = (witers + 1) // 2
    npass = (n + _RNG - 1) // _RNG
    n_out = npass * _RNG
    rows_t = _RNG // 16

    @functools.partial(
        pl.kernel, mesh=mesh,
        out_type=jax.ShapeDtypeStruct((2, n_out, _WID), jnp.float32),
        scratch_types=[pltpu.VMEM((_CHS,), jnp.int32),
                       pltpu.VMEM((_CHS,), jnp.int32),
                       pltpu.VMEM((_CHS,), jnp.int32),
                       pltpu.VMEM((_CHS,), jnp.int32),
                       pltpu.VMEM((_CHS, _WID), jnp.float32),
                       pltpu.VMEM((_CHS, _WID), jnp.float32),
                       pltpu.SemaphoreType.DMA,
                       pltpu.SemaphoreType.DMA,
                       pltpu.VMEM_SHARED((_RNG + 8, _WID), jnp.float32)],
    )
    def sk(wvw_hbm, dst_hbm, z_hbm, nump_o,
           ia, ib, ja, jb, ba, bb, sa, sb, acc):
        c = lax.axis_index("c")
        s = lax.axis_index("s")
        w = s * 2 + c
        r0 = pl.multiple_of(s * rows_t, 8)
        sets = ((ia, ja, ba, sa), (ib, jb, bb, sb))

        def issue(jj, st):
            chunk = jj * _NW + w

            @pl.when(chunk < chunks)
            def _():
                base = pl.multiple_of(chunk * _CHS, 8)
                pltpu.async_copy(dst_hbm.at[pl.ds(base, _CHS)], st[0], st[3])
                pltpu.async_copy(wvw_hbm.at[pl.ds(base, _CHS)], st[2], st[3])

        for p in range(npass):
            lo = p * _RNG
            pltpu.sync_copy(z_hbm.at[pl.ds(r0, rows_t)],
                            acc.at[pl.ds(r0, rows_t)])

            @pl.when(s == 0)
            def _():
                pltpu.sync_copy(z_hbm.at[pl.ds(_RNG, 8)],
                                acc.at[pl.ds(_RNG, 8)])

            plsc.subcore_barrier()
            issue(0, sets[0])
            issue(1, sets[1])

            def lbody(t, carry, lo=lo):
                for b2 in range(2):
                    st = sets[b2]
                    jj = t * 2 + b2
                    chunk = jj * _NW + w

                    @pl.when(chunk < chunks)
                    def _(st=st, jj=jj, chunk=chunk, lo=lo):
                        base = pl.multiple_of(chunk * _CHS, 8)
                        pltpu.make_async_copy(
                            dst_hbm.at[pl.ds(base, _CHS)], st[0], st[3]).wait()
                        pltpu.make_async_copy(
                            wvw_hbm.at[pl.ds(base, _CHS)], st[2], st[3]).wait()

                        def vbody(j, cc):
                            o = pl.multiple_of(j * 16, 8)
                            rel = st[0][pl.ds(o, 16)] - lo
                            ok = (rel >= 0) & (rel < _RNG)
                            st[1][pl.ds(o, 16)] = jnp.where(ok, rel, _RNG)
                            return cc

                        lax.fori_loop(0, _CHS // 16, vbody, 0)
                        pltpu.sync_copy(st[2], acc.at[st[1]], add=True)

                    issue(jj + 2, st)
                return carry

            lax.fori_loop(0, pairs, lbody, 0)
            plsc.subcore_barrier()
            pltpu.sync_copy(acc.at[pl.ds(r0, rows_t)],
                            nump_o.at[c, pl.ds(pl.multiple_of(lo + r0, 8),
                                               rows_t)])

    return sk(wvw, dst, zeros)


def _embed_h(h8, w8, b, pos_enc, n):
    return pl.pallas_call(
        _embed_h_body,
        grid=(n // _NBLK,),
        in_specs=[pl.BlockSpec((_NBLK, 8), lambda i: (i, 0)),
                  _fullspec(w8.shape), _fullspec(b.shape),
                  pl.BlockSpec((_NBLK, _HID), lambda i: (i, 0))],
        out_specs=pl.BlockSpec((_NBLK, _HID), lambda i: (i, 0)),
        out_shape=jax.ShapeDtypeStruct((n, _HID), jnp.float32),
    )(h8, w8, b, pos_enc)


def _embed_e(e8, w8, b, m):
    return pl.pallas_call(
        _embed_e_body,
        grid=(m // _EBLK,),
        in_specs=[pl.BlockSpec((_EBLK, 8), lambda i: (i, 0)),
                  _fullspec(w8.shape), _fullspec(b.shape)],
        out_specs=pl.BlockSpec((_EBLK, _HID), lambda i: (i, 0)),
        out_shape=jax.ShapeDtypeStruct((m, _HID), jnp.float32),
    )(e8, w8, b)


def _qkv(hh, lp, n):
    args = [hh,
            _padc(lp['Q'][0]), _padc(_row1(lp['Q'][1])),
            _padc(lp['K'][0]), _padc(_row1(lp['K'][1])),
            _padc(lp['V'][0]), _padc(_row1(lp['V'][1]))]
    blk = pl.BlockSpec((_NBLK, _HID), lambda i: (i, 0))
    blkw = pl.BlockSpec((_NBLK, _WID), lambda i: (i, 0))
    return pl.pallas_call(
        _qkv_body,
        grid=(n // _NBLK,),
        in_specs=[blk] + [_fullspec(a.shape) for a in args[1:]],
        out_specs=(blkw, blkw, blkw),
        out_shape=tuple(jax.ShapeDtypeStruct((n, _WID), jnp.float32)
                        for _ in range(3)),
    )(*args)


def _edge_stage(ee, ke, qe, ve, lp, consts, m):
    mh, mt, pw = consts
    args = [ee, ke, qe, ve,
            _padc(lp['Ep'][0]), _padc(_row1(lp['Ep'][1])),
            _padr(lp['Oe'][0]), _row1(lp['Oe'][1]),
            lp['ffn_e1'][0], _row1(lp['ffn_e1'][1]),
            lp['ffn_e2'][0], _row1(lp['ffn_e2'][1]),
            _row1(lp['ln1_e'][0]), _row1(lp['ln1_e'][1]),
            _row1(lp['ln2_e'][0]), _row1(lp['ln2_e'][1]),
            mh, mt, pw]
    blk = pl.BlockSpec((_EBLK, _HID), lambda i: (i, 0))
    blkw = pl.BlockSpec((_EBLK, _WID), lambda i: (i, 0))
    return pl.pallas_call(
        _edge_body,
        grid=(m // _EBLK,),
        in_specs=[blk, blkw, blkw, blkw] + [_fullspec(a.shape) for a in args[4:]],
        out_specs=(blk, blkw),
        out_shape=(jax.ShapeDtypeStruct((m, _HID), jnp.float32),
                   jax.ShapeDtypeStruct((m, _WID), jnp.float32)),
    )(*args)


def _node_stage(hh, nump, lp, sel96, mt8, n):
    s = nump.shape[0]
    args = [hh, nump, sel96, mt8,
            lp['O'][0], _row1(lp['O'][1]),
            lp['ffn_h1'][0], _row1(lp['ffn_h1'][1]),
            lp['ffn_h2'][0], _row1(lp['ffn_h2'][1]),
            _row1(lp['ln1_h'][0]), _row1(lp['ln1_h'][1]),
            _row1(lp['ln2_h'][0]), _row1(lp['ln2_h'][1])]
    blk = pl.BlockSpec((_NBLK, _HID), lambda i: (i, 0))
    return pl.pallas_call(
        _node_body,
        grid=(n // _NBLK,),
        in_specs=[blk,
                  pl.BlockSpec((s, _NBLK, _WID), lambda i: (0, i, 0))]
                 + [_fullspec(a.shape) for a in args[2:]],
        out_specs=blk,
        out_shape=jax.ShapeDtypeStruct((n, _HID), jnp.float32),
    )(*args)


def _readout(hh, batch, params, o96, n):
    batch3 = batch.reshape(n // _NBLK, 1, _NBLK)
    sums, cnts = pl.pallas_call(
        _pool_body,
        grid=(n // _NBLK,),
        in_specs=[pl.BlockSpec((1, 1, _NBLK), lambda i: (i, 0, 0)),
                  pl.BlockSpec((_NBLK, _HID), lambda i: (i, 0))],
        out_specs=(pl.BlockSpec((_NG, _HID), lambda i: (0, 0)),
                   pl.BlockSpec((_NG, 8), lambda i: (0, 0))),
        out_shape=(jax.ShapeDtypeStruct((_NG, _HID), jnp.float32),
                   jax.ShapeDtypeStruct((_NG, 8), jnp.float32)),
    )(batch3, hh)
    args = [sums, cnts, o96,
            params['mlp1'][0], _row1(params['mlp1'][1]),
            params['mlp2'][0], _row1(params['mlp2'][1]),
            params['mlp3'][0], _row1(params['mlp3'][1])]
    return pl.pallas_call(
        _mlp_body,
        in_specs=[_fullspec(a.shape) for a in args],
        out_specs=_fullspec((_NG, _OUT)),
        out_shape=jax.ShapeDtypeStruct((_NG, _OUT), jnp.float32),
    )(*args)


def kernel(h, e, pos_enc, params, edge_index, batch):
    n = h.shape[0]
    m = e.shape[0]
    src = edge_index[0]
    dst = edge_index[1]

    mh = jnp.asarray(_MH128)
    mt = jnp.asarray(_MT128)
    pw = jnp.asarray(_PW)
    sel96 = jnp.asarray(_SEL96)
    mt8 = jnp.asarray(_MT8)
    o96 = jnp.asarray(_O96)

    h8 = jnp.pad(h, ((0, 0), (0, 1)))
    e8 = jnp.pad(e, ((0, 0), (0, 1)))
    wh8 = jnp.pad(params['emb_h'][0], ((0, 1), (0, 0)))
    we8 = jnp.pad(params['emb_e'][0], ((0, 1), (0, 0)))

    hh = _embed_h(h8, wh8, _row1(params['emb_h'][1]), pos_enc, n)
    ee = _embed_e(e8, we8, _row1(params['emb_e'][1]), m)

    zeros = jnp.zeros((_RNG + 8, _WID), jnp.float32)
    for lp in params['layers']:
        q, k, v = _qkv(hh, lp, n)
        ke, qe, ve = _sc_gather(k, q, v, src, dst, n, m)
        ee, wvw = _edge_stage(ee, ke, qe, ve, lp, (mh, mt, pw), m)
        nump = _sc_scatter(wvw, dst, zeros, n, m)
        hh = _node_stage(hh, nump, lp, sel96, mt8, n)

    return _readout(hh, batch, params, o96, n)


# speedup vs baseline: 39.6365x; 1.0197x over previous
import functools
import numpy as np
import jax
import jax.numpy as jnp
from jax import lax
from jax.experimental import pallas as pl
from jax.experimental.pallas import tpu as pltpu
from jax.experimental.pallas import tpu_sc as plsc

_HID = 96
_HEADS = 8
_DK = 12
_NG = 256
_OUT = 37
_EBLK = 2000
_NBLK = 1000
_ISQ = float(1.0 / np.sqrt(_DK))

_WID = 128
_CHG = 800
_CHS = 160
_NW = 32
_RNG = 11008

_MH = np.kron(np.eye(_HEADS), np.ones((_DK, 1))).astype(np.float32)
_MT = np.kron(np.eye(_HEADS), np.ones((1, _DK))).astype(np.float32)
_MH128 = np.pad(_MH, ((0, _WID - _HID), (0, 0)))
_MT128 = np.pad(_MT, ((0, 0), (0, _WID - _HID)))
_PW = np.zeros((_HEADS, _WID), np.float32)
for _h in range(_HEADS):
    _PW[_h, _HID + _h] = 1.0
_SEL96 = np.pad(np.eye(_HID, dtype=np.float32), ((0, _WID - _HID), (0, 0)))
_MT8 = np.zeros((_WID, _HID), np.float32)
for _h in range(_HEADS):
    _MT8[_HID + _h, _h * _DK:(_h + 1) * _DK] = 1.0
_O96 = (np.ones((8, _HID)) / 8.0).astype(np.float32)


def _fullspec(shape):
    nd = len(shape)
    return pl.BlockSpec(shape, lambda *_: (0,) * nd)


def _dot(a, b):
    return jnp.dot(a, b, preferred_element_type=jnp.float32)


def _ln(x, g, b):
    m = jnp.mean(x, axis=-1, keepdims=True)
    xc = x - m
    v = jnp.mean(xc * xc, axis=-1, keepdims=True)
    return xc * lax.rsqrt(v + 1e-5) * g + b


def _row1(b):
    return b.reshape(1, -1)


def _padc(w):
    return jnp.pad(w, ((0, 0), (0, _WID - w.shape[1])))


def _padr(w):
    return jnp.pad(w, ((0, _WID - w.shape[0]), (0, 0)))


def _embed_h_body(x_ref, w_ref, b_ref, pe_ref, o_ref):
    o_ref[...] = _dot(x_ref[...], w_ref[...]) + b_ref[...] + pe_ref[...]


def _embed_e_body(x_ref, w_ref, b_ref, o_ref):
    o_ref[...] = _dot(x_ref[...], w_ref[...]) + b_ref[...]


def _qkv_body(x_ref, wq, bq, wk, bk, wv, bv, q_o, k_o, v_o):
    x = x_ref[...]
    q_o[...] = _dot(x, wq[...]) + bq[...]
    k_o[...] = _dot(x, wk[...]) + bk[...]
    v_o[...] = _dot(x, wv[...]) + bv[...]


def _edge_body(ee_ref, ke_ref, qe_ref, ve_ref,
               wep, bep, woe, boe, w1, b1, w2, b2,
               g1, gb1, g2, gb2, mh, mt, pw,
               ee_o, wvw_o):
    ee = ee_ref[...]
    ep = _dot(ee, wep[...]) + bep[...]
    score = ke_ref[...] * qe_ref[...] * ep * _ISQ
    s = _dot(score, mh[...])
    w = jnp.exp(jnp.clip(s, -5.0, 5.0))
    wvw_o[...] = _dot(w, mt[...]) * ve_ref[...] + _dot(w, pw[...])
    oe = _dot(score, woe[...]) + boe[...]
    e1 = _ln(ee + oe, g1[...], gb1[...])
    f = jnp.maximum(_dot(e1, w1[...]) + b1[...], 0.0)
    f = _dot(f, w2[...]) + b2[...]
    ee_o[...] = _ln(e1 + f, g2[...], gb2[...])


def _node_body(hh_ref, nump_ref, sel96, mt8, wo, bo, w1, b1, w2, b2,
               g1, gb1, g2, gb2, out_ref):
    nd = jnp.sum(nump_ref[...], axis=0)
    num = _dot(nd, sel96[...])
    denf = _dot(nd, mt8[...])
    h_attn = num / (denf + 1e-6)
    hh = hh_ref[...]
    o = _dot(h_attn, wo[...]) + bo[...]
    h1 = _ln(hh + o, g1[...], gb1[...])
    f = jnp.maximum(_dot(h1, w1[...]) + b1[...], 0.0)
    f = _dot(f, w2[...]) + b2[...]
    out_ref[...] = _ln(h1 + f, g2[...], gb2[...])


def _pool_body(batch_ref, hh_ref, sums_ref, cnts_ref):
    i = pl.program_id(0)

    @pl.when(i == 0)
    def _():
        sums_ref[...] = jnp.zeros_like(sums_ref)
        cnts_ref[...] = jnp.zeros_like(cnts_ref)

    b = batch_ref[0, 0, :]
    oh = (b[:, None] == lax.broadcasted_iota(jnp.int32, (_NBLK, _NG), 1))
    oh = oh.astype(jnp.float32)
    sums_ref[...] += lax.dot_general(oh, hh_ref[...], (((0,), (0,)), ((), ())),
                                     preferred_element_type=jnp.float32)
    cnts_ref[...] += lax.dot_general(oh, jnp.ones((_NBLK, 8), jnp.float32),
                                     (((0,), (0,)), ((), ())),
                                     preferred_element_type=jnp.float32)


def _mlp_body(sums_ref, cnts_ref, o96, w1, b1, w2, b2, w3, b3, y_ref):
    cnt96 = _dot(cnts_ref[...], o96[...])
    hg = sums_ref[...] / jnp.maximum(cnt96, 1.0)
    y = jnp.maximum(_dot(hg, w1[...]) + b1[...], 0.0)
    y = jnp.maximum(_dot(y, w2[...]) + b2[...], 0.0)
    y_ref[...] = _dot(y, w3[...]) + b3[...]


def _sc_gather(k, q, v, src, dst, n, m):
    mesh = plsc.VectorSubcoreMesh(core_axis_name="c", subcore_axis_name="s")
    chunks = m // _CHG
    witers = (chunks + _NW - 1) // _NW

    @functools.partial(
        pl.kernel, mesh=mesh,
        out_type=(jax.ShapeDtypeStruct((m, _WID), jnp.float32),
                  jax.ShapeDtypeStruct((m, _WID), jnp.float32),
                  jax.ShapeDtypeStruct((m, _WID), jnp.float32)),
        scratch_types=[pltpu.VMEM((_CHG,), jnp.int32),
                       pltpu.VMEM((_CHG, _WID), jnp.float32),
                       pltpu.SemaphoreType.DMA],
    )
    def gk(k_hbm, q_hbm, v_hbm, src_hbm, dst_hbm, ke_o, qe_o, ve_o,
           idx_v, rows_v, sem):
        c = lax.axis_index("c")
        s = lax.axis_index("s")
        w = s * 2 + c

        def body(t, carry):
            chunk = t * _NW + w

            @pl.when(chunk < chunks)
            def _():
                base = pl.multiple_of(chunk * _CHG, 8)
                pltpu.sync_copy(src_hbm.at[pl.ds(base, _CHG)], idx_v)
                pltpu.async_copy(k_hbm.at[idx_v], rows_v, sem).wait()
                pltpu.sync_copy(rows_v, ke_o.at[pl.ds(base, _CHG)])
                pltpu.async_copy(v_hbm.at[idx_v], rows_v, sem).wait()
                pltpu.sync_copy(rows_v, ve_o.at[pl.ds(base, _CHG)])
                pltpu.sync_copy(dst_hbm.at[pl.ds(base, _CHG)], idx_v)
                pltpu.async_copy(q_hbm.at[idx_v], rows_v, sem).wait()
                pltpu.sync_copy(rows_v, qe_o.at[pl.ds(base, _CHG)])

            return carry

        lax.fori_loop(0, witers, body, 0)

    return gk(k, q, v, src, dst)


def _sc_scatter(wvw, dst, zeros, n, m):
    mesh = plsc.VectorSubcoreMesh(core_axis_name="c", subcore_axis_name="s")
    chunks = m // _CHS
    witers = (chunks + _NW - 1) // _NW
    pairs = (witers + 1) // 2
    npass = (n + _RNG - 1) // _RNG
    n_out = npass * _RNG
    rows_t = _RNG // 16

    @functools.partial(
        pl.kernel, mesh=mesh,
        out_type=jax.ShapeDtypeStruct((2, n_out, _WID), jnp.float32),
        scratch_types=[pltpu.VMEM((_CHS,), jnp.int32),
                       pltpu.VMEM((_CHS,), jnp.int32),
                       pltpu.VMEM((_CHS,), jnp.int32),
                       pltpu.VMEM((_CHS,), jnp.int32),
                       pltpu.VMEM((_CHS, _WID), jnp.float32),
                       pltpu.VMEM((_CHS, _WID), jnp.float32),
                       pltpu.SemaphoreType.DMA,
                       pltpu.SemaphoreType.DMA,
                       pltpu.VMEM_SHARED((_RNG + 8, _WID), jnp.float32)],
    )
    def sk(wvw_hbm, dst_hbm, z_hbm, nump_o,
           ia, ib, ja, jb, ba, bb, sa, sb, acc):
        c = lax.axis_index("c")
        s = lax.axis_index("s")
        w = s * 2 + c
        r0 = pl.multiple_of(s * rows_t, 8)
        sets = ((ia, ja, ba, sa), (ib, jb, bb, sb))

        def issue(jj, st):
            chunk = jj * _NW + w

            @pl.when(chunk < chunks)
            def _():
                base = pl.multiple_of(chunk * _CHS, 8)
                pltpu.async_copy(dst_hbm.at[pl.ds(base, _CHS)], st[0], st[3])
                pltpu.async_copy(wvw_hbm.at[pl.ds(base, _CHS)], st[2], st[3])

        for p in range(npass):
            lo = p * _RNG
            pltpu.sync_copy(z_hbm.at[pl.ds(r0, rows_t)],
                            acc.at[pl.ds(r0, rows_t)])

            @pl.when(s == 0)
            def _():
                pltpu.sync_copy(z_hbm.at[pl.ds(_RNG, 8)],
                                acc.at[pl.ds(_RNG, 8)])

            plsc.subcore_barrier()
            issue(0, sets[0])
            issue(1, sets[1])

            def lbody(t, carry, lo=lo):
                for b2 in range(2):
                    st = sets[b2]
                    jj = t * 2 + b2
                    chunk = jj * _NW + w

                    @pl.when(chunk < chunks)
                    def _(st=st, jj=jj, chunk=chunk, lo=lo):
                        base = pl.multiple_of(chunk * _CHS, 8)
                        pltpu.make_async_copy(
                            dst_hbm.at[pl.ds(base, _CHS)], st[0], st[3]).wait()
                        pltpu.make_async_copy(
                            wvw_hbm.at[pl.ds(base, _CHS)], st[2], st[3]).wait()

                        def vbody(j, cc):
                            o = pl.multiple_of(j * 16, 8)
                            rel = st[0][pl.ds(o, 16)] - lo
                            ok = (rel >= 0) & (rel < _RNG)
                            st[1][pl.ds(o, 16)] = jnp.where(ok, rel, _RNG)
                            return cc

                        lax.fori_loop(0, _CHS // 16, vbody, 0)
                        pltpu.sync_copy(st[2], acc.at[st[1]], add=True)

                    issue(jj + 2, st)
                return carry

            lax.fori_loop(0, pairs, lbody, 0)
            plsc.subcore_barrier()
            pltpu.sync_copy(acc.at[pl.ds(r0, rows_t)],
                            nump_o.at[c, pl.ds(pl.multiple_of(lo + r0, 8),
                                               rows_t)])

    return sk(wvw, dst, zeros)


def _embed_h(h8, w8, b, pos_enc, n):
    return pl.pallas_call(
        _embed_h_body,
        grid=(n // _NBLK,),
        in_specs=[pl.BlockSpec((_NBLK, 8), lambda i: (i, 0)),
                  _fullspec(w8.shape), _fullspec(b.shape),
                  pl.BlockSpec((_NBLK, _HID), lambda i: (i, 0))],
        out_specs=pl.BlockSpec((_NBLK, _HID), lambda i: (i, 0)),
        out_shape=jax.ShapeDtypeStruct((n, _HID), jnp.float32),
    )(h8, w8, b, pos_enc)


def _embed_e(e8, w8, b, m):
    return pl.pallas_call(
        _embed_e_body,
        grid=(m // _EBLK,),
        in_specs=[pl.BlockSpec((_EBLK, 8), lambda i: (i, 0)),
                  _fullspec(w8.shape), _fullspec(b.shape)],
        out_specs=pl.BlockSpec((_EBLK, _HID), lambda i: (i, 0)),
        out_shape=jax.ShapeDtypeStruct((m, _HID), jnp.float32),
    )(e8, w8, b)


def _qkv(hh, lp, n):
    args = [hh,
            _padc(lp['Q'][0]), _padc(_row1(lp['Q'][1])),
            _padc(lp['K'][0]), _padc(_row1(lp['K'][1])),
            _padc(lp['V'][0]), _padc(_row1(lp['V'][1]))]
    blk = pl.BlockSpec((_NBLK, _HID), lambda i: (i, 0))
    blkw = pl.BlockSpec((_NBLK, _WID), lambda i: (i, 0))
    return pl.pallas_call(
        _qkv_body,
        grid=(n // _NBLK,),
        in_specs=[blk] + [_fullspec(a.shape) for a in args[1:]],
        out_specs=(blkw, blkw, blkw),
        out_shape=tuple(jax.ShapeDtypeStruct((n, _WID), jnp.float32)
                        for _ in range(3)),
    )(*args)


def _edge_stage(ee, ke, qe, ve, lp, consts, m):
    mh, mt, pw = consts
    args = [ee, ke, qe, ve,
            _padc(lp['Ep'][0]), _padc(_row1(lp['Ep'][1])),
            _padr(lp['Oe'][0]), _row1(lp['Oe'][1]),
            lp['ffn_e1'][0], _row1(lp['ffn_e1'][1]),
            lp['ffn_e2'][0], _row1(lp['ffn_e2'][1]),
            _row1(lp['ln1_e'][0]), _row1(lp['ln1_e'][1]),
            _row1(lp['ln2_e'][0]), _row1(lp['ln2_e'][1]),
            mh, mt, pw]
    blk = pl.BlockSpec((_EBLK, _HID), lambda i: (i, 0))
    blkw = pl.BlockSpec((_EBLK, _WID), lambda i: (i, 0))
    return pl.pallas_call(
        _edge_body,
        grid=(m // _EBLK,),
        in_specs=[blk, blkw, blkw, blkw] + [_fullspec(a.shape) for a in args[4:]],
        out_specs=(blk, blkw),
        out_shape=(jax.ShapeDtypeStruct((m, _HID), jnp.float32),
                   jax.ShapeDtypeStruct((m, _WID), jnp.float32)),
    )(*args)


def _node_stage(hh, nump, lp, sel96, mt8, n):
    s = nump.shape[0]
    args = [hh, nump, sel96, mt8,
            lp['O'][0], _row1(lp['O'][1]),
            lp['ffn_h1'][0], _row1(lp['ffn_h1'][1]),
            lp['ffn_h2'][0], _row1(lp['ffn_h2'][1]),
            _row1(lp['ln1_h'][0]), _row1(lp['ln1_h'][1]),
            _row1(lp['ln2_h'][0]), _row1(lp['ln2_h'][1])]
    blk = pl.BlockSpec((_NBLK, _HID), lambda i: (i, 0))
    return pl.pallas_call(
        _node_body,
        grid=(n // _NBLK,),
        in_specs=[blk,
                  pl.BlockSpec((s, _NBLK, _WID), lambda i: (0, i, 0))]
                 + [_fullspec(a.shape) for a in args[2:]],
        out_specs=blk,
        out_shape=jax.ShapeDtypeStruct((n, _HID), jnp.float32),
    )(*args)


def _readout(hh, batch, params, o96, n):
    batch3 = batch.reshape(n // _NBLK, 1, _NBLK)
    sums, cnts = pl.pallas_call(
        _pool_body,
        grid=(n // _NBLK,),
        in_specs=[pl.BlockSpec((1, 1, _NBLK), lambda i: (i, 0, 0)),
                  pl.BlockSpec((_NBLK, _HID), lambda i: (i, 0))],
        out_specs=(pl.BlockSpec((_NG, _HID), lambda i: (0, 0)),
                   pl.BlockSpec((_NG, 8), lambda i: (0, 0))),
        out_shape=(jax.ShapeDtypeStruct((_NG, _HID), jnp.float32),
                   jax.ShapeDtypeStruct((_NG, 8), jnp.float32)),
    )(batch3, hh)
    args = [sums, cnts, o96,
            params['mlp1'][0], _row1(params['mlp1'][1]),
            params['mlp2'][0], _row1(params['mlp2'][1]),
            params['mlp3'][0], _row1(params['mlp3'][1])]
    return pl.pallas_call(
        _mlp_body,
        in_specs=[_fullspec(a.shape) for a in args],
        out_specs=_fullspec((_NG, _OUT)),
        out_shape=jax.ShapeDtypeStruct((_NG, _OUT), jnp.float32),
    )(*args)


def kernel(h, e, pos_enc, params, edge_index, batch):
    n = h.shape[0]
    m = e.shape[0]
    src = edge_index[0]
    dst = edge_index[1]

    mh = jnp.asarray(_MH128)
    mt = jnp.asarray(_MT128)
    pw = jnp.asarray(_PW)
    sel96 = jnp.asarray(_SEL96)
    mt8 = jnp.asarray(_MT8)
    o96 = jnp.asarray(_O96)

    h8 = jnp.pad(h, ((0, 0), (0, 1)))
    e8 = jnp.pad(e, ((0, 0), (0, 1)))
    wh8 = jnp.pad(params['emb_h'][0], ((0, 1), (0, 0)))
    we8 = jnp.pad(params['emb_e'][0], ((0, 1), (0, 0)))

    hh = _embed_h(h8, wh8, _row1(params['emb_h'][1]), pos_enc, n)
    ee = _embed_e(e8, we8, _row1(params['emb_e'][1]), m)

    zeros = jnp.zeros((_RNG + 8, _WID), jnp.float32)
    for lp in params['layers']:
        q, k, v = _qkv(hh, lp, n)
        ke, qe, ve = _sc_gather(k, q, v, src, dst, n, m)
        ee, wvw = _edge_stage(ee, ke, qe, ve, lp, (mh, mt, pw), m)
        nump = _sc_scatter(wvw, dst, zeros, n, m)
        hh = _node_stage(hh, nump, lp, sel96, mt8, n)

    return _readout(hh, batch, params, o96, n)
